# baseline (device time: 224502 ns/iter reference)
import jax
import jax.numpy as jnp
from jax import lax
from jax.experimental import pallas as pl
from jax.experimental.pallas import tpu as pltpu

N_DEV = 4
B_LOC = 2
SQ = 512
NG = 4
GS = 128
HQ_LOC = 8
DH = 64
D_MODEL = 768
D_HEADS = HQ_LOC * DH
ROWS = B_LOC * NG * GS


def _body(x_ref, wq_ref, kg_ref, vg_ref, wo_ref, out_ref,
          wq_comm, wo_comm, q_scr, ctx_scr,
          sq_send, sq_recv, so_send, so_recv):
    my = lax.axis_index("i")
    left = lax.rem(my + (N_DEV - 1), N_DEV)
    right = lax.rem(my + 1, N_DEV)

    barrier_sem = pltpu.get_barrier_semaphore()
    for nbr in (left, right):
        pl.semaphore_signal(barrier_sem, inc=1, device_id=(nbr,),
                            device_id_type=pl.DeviceIdType.MESH)
    pl.semaphore_wait(barrier_sem, 2)

    def compute_accum(j, wq, wo, is_first):
        q_scr[...] = jnp.dot(x_ref[...], wq, preferred_element_type=jnp.float32)

        def bc_body(bc, carry):
            r0 = bc * GS
            for h in range(HQ_LOC):
                c0 = h * DH
                q = q_scr[pl.ds(r0, GS), c0:c0 + DH]
                k = kg_ref[j, pl.ds(r0, GS), c0:c0 + DH]
                v = vg_ref[j, pl.ds(r0, GS), c0:c0 + DH]
                s = lax.dot_general(
                    q, k, (((1,), (1,)), ((), ())),
                    preferred_element_type=jnp.float32) * 0.125
                m = jnp.max(s, axis=1, keepdims=True)
                e = jnp.exp(s - m)
                p = e / jnp.sum(e, axis=1, keepdims=True)
                ctx_scr[pl.ds(r0, GS), c0:c0 + DH] = jnp.dot(
                    p, v, preferred_element_type=jnp.float32)
            return carry

        lax.fori_loop(0, B_LOC * NG, bc_body, 0)

        contrib = jnp.dot(ctx_scr[...], wo,
                          preferred_element_type=jnp.float32)
        if is_first:
            out_ref[...] = contrib
        else:
            out_ref[...] = out_ref[...] + contrib

    descs = []
    rq = pltpu.make_async_remote_copy(
        src_ref=wq_ref, dst_ref=wq_comm.at[0],
        send_sem=sq_send.at[0], recv_sem=sq_recv.at[0],
        device_id=(right,), device_id_type=pl.DeviceIdType.MESH)
    ro = pltpu.make_async_remote_copy(
        src_ref=wo_ref, dst_ref=wo_comm.at[0],
        send_sem=so_send.at[0], recv_sem=so_recv.at[0],
        device_id=(right,), device_id_type=pl.DeviceIdType.MESH)
    rq.start()
    ro.start()
    descs.append((rq, ro))

    compute_accum(my, wq_ref[...], wo_ref[...], is_first=True)

    for h in range(1, N_DEV):
        slot = h - 1
        rq_prev, ro_prev = descs[slot]
        rq_prev.wait_recv()
        ro_prev.wait_recv()
        if h < N_DEV - 1:
            rq = pltpu.make_async_remote_copy(
                src_ref=wq_comm.at[slot], dst_ref=wq_comm.at[h],
                send_sem=sq_send.at[h], recv_sem=sq_recv.at[h],
                device_id=(right,), device_id_type=pl.DeviceIdType.MESH)
            ro = pltpu.make_async_remote_copy(
                src_ref=wo_comm.at[slot], dst_ref=wo_comm.at[h],
                send_sem=so_send.at[h], recv_sem=so_recv.at[h],
                device_id=(right,), device_id_type=pl.DeviceIdType.MESH)
            rq.start()
            ro.start()
            descs.append((rq, ro))
        j = lax.rem(my + (N_DEV - h), N_DEV)
        compute_accum(j, wq_comm[slot], wo_comm[slot], is_first=False)

    for rq_d, ro_d in descs:
        rq_d.wait_send()
        ro_d.wait_send()


def kernel(x, Wq, K_ext, V_ext, Wo):
    my = lax.axis_index("i")

    k_loc = lax.dynamic_slice_in_dim(K_ext, my * B_LOC, B_LOC, axis=0)
    v_loc = lax.dynamic_slice_in_dim(V_ext, my * B_LOC, B_LOC, axis=0)

    xg = (x.reshape(B_LOC, 2, NG, 64, D_MODEL)
           .transpose(0, 2, 1, 3, 4)
           .reshape(ROWS, D_MODEL))

    def prep(t):
        return (t.reshape(B_LOC, 2, NG, 64, N_DEV, HQ_LOC, DH)
                 .transpose(4, 0, 2, 1, 3, 5, 6)
                 .reshape(N_DEV, ROWS, D_HEADS))

    kg = prep(k_loc)
    vg = prep(v_loc)

    outg = pl.pallas_call(
        _body,
        out_shape=jax.ShapeDtypeStruct((ROWS, D_MODEL), jnp.float32),
        in_specs=[pl.BlockSpec(memory_space=pltpu.VMEM)] * 5,
        out_specs=pl.BlockSpec(memory_space=pltpu.VMEM),
        scratch_shapes=[
            pltpu.VMEM((N_DEV - 1, D_MODEL, D_HEADS), jnp.float32),
            pltpu.VMEM((N_DEV - 1, D_HEADS, D_MODEL), jnp.float32),
            pltpu.VMEM((ROWS, D_HEADS), jnp.float32),
            pltpu.VMEM((ROWS, D_HEADS), jnp.float32),
            pltpu.SemaphoreType.DMA((N_DEV - 1,)),
            pltpu.SemaphoreType.DMA((N_DEV - 1,)),
            pltpu.SemaphoreType.DMA((N_DEV - 1,)),
            pltpu.SemaphoreType.DMA((N_DEV - 1,)),
        ],
        compiler_params=pltpu.CompilerParams(collective_id=0),
    )(xg, Wq, kg, vg, Wo)

    return (outg.reshape(B_LOC, NG, 2, 64, D_MODEL)
                .transpose(0, 2, 1, 3, 4)
                .reshape(B_LOC, SQ, D_MODEL))
